# Initial kernel scaffold; baseline (speedup 1.0000x reference)
import functools

import jax
import jax.numpy as jnp
from jax import lax
from jax.experimental import pallas as pl
from jax.experimental.pallas import tpu as pltpu
from jax.experimental.pallas import tpu_sc as plsc

N = 10000
E = 320000
T = 8
C = 128
NP = 10240
NC = 2
NS = 16
RPT = NP // NS
TPS = T // NC

EPT = E // NS
K = 32
NCH = EPT // K

EPW = E // (NC * NS)
DK = 80
DCH = EPW // DK
DL = 16

ZR = 160

_f32 = jnp.float32
_i32 = jnp.int32

_sc_mesh = plsc.VectorSubcoreMesh(
    core_axis_name="c", subcore_axis_name="s", num_cores=NC, num_subcores=NS)


@functools.partial(
    pl.kernel,
    out_type=jax.ShapeDtypeStruct((NC * NP, DL), _f32),
    mesh=_sc_mesh,
    scratch_types=[
        pltpu.VMEM((DCH, DK), _i32),
        pltpu.VMEM((DCH, DK), _f32),
        pltpu.VMEM((DK, DL), _f32),
        pltpu.VMEM((RPT, DL), _f32),
        pltpu.VMEM_SHARED((NP, DL), _f32),
    ],
)
def _deg_kernel(col3, w3, out, colstage, wstage, dbuf, zbuf, dacc):
    cid = lax.axis_index("c")
    sid = lax.axis_index("s")
    wid = cid * NS + sid
    pltpu.sync_copy(col3.at[wid], colstage)
    pltpu.sync_copy(w3.at[wid], wstage)

    zv = jnp.zeros((DL,), _f32)

    def zb(i, carry):
        zbuf[i, :] = zv
        return carry

    lax.fori_loop(0, RPT, zb, 0)
    pltpu.sync_copy(zbuf, dacc.at[pl.ds(sid * RPT, RPT)])
    plsc.subcore_barrier()

    def chunk(j, carry):
        for jj in range(DK):
            dbuf[jj, :] = jnp.full((DL,), wstage[j, jj], _f32)
        pltpu.sync_copy(dbuf, dacc.at[colstage.at[j]], add=True)
        return carry

    lax.fori_loop(0, DCH, chunk, 0)
    plsc.subcore_barrier()
    pltpu.sync_copy(dacc.at[pl.ds(sid * RPT, RPT)],
                    out.at[pl.ds(cid * NP + sid * RPT, RPT)])


@functools.partial(
    pl.kernel,
    out_type=jax.ShapeDtypeStruct((T * NP, C), _f32),
    mesh=_sc_mesh,
    scratch_types=[
        pltpu.VMEM((EPT,), _i32),
        pltpu.VMEM((EPT,), _f32),
        pltpu.VMEM((NCH, K), _i32),
        pltpu.VMEM((K,), _i32),
        pltpu.VMEM((K, C), _f32),
        pltpu.VMEM((ZR, C), _f32),
        pltpu.VMEM_SHARED((NP, C), _f32),
        pltpu.SemaphoreType.DMA,
    ],
)
def _spmm_kernel(hpp, rowv, col3, wv, out,
                 rowslab, wslab, colstage, idxbuf, gbuf, zbuf, acc, sem):
    cid = lax.axis_index("c")
    sid = lax.axis_index("s")
    pltpu.sync_copy(rowv.at[pl.ds(sid * EPT, EPT)], rowslab)
    pltpu.sync_copy(wv.at[pl.ds(sid * EPT, EPT)], wslab)
    pltpu.sync_copy(col3.at[sid], colstage)

    zv = jnp.zeros((16,), _f32)

    def zb(i, carry):
        for v in range(C // 16):
            zbuf[i, pl.ds(v * 16, 16)] = zv
        return carry

    lax.fori_loop(0, ZR, zb, 0)

    for tl in range(TPS):
        t = cid * TPS + tl
        toff = t * NP

        def zc(i, carry):
            pltpu.sync_copy(zbuf, acc.at[pl.ds(sid * RPT + i * ZR, ZR)])
            return carry

        lax.fori_loop(0, RPT // ZR, zc, 0)
        plsc.subcore_barrier()

        def chunk(cix, carry):
            base = cix * K
            r0 = rowslab[pl.ds(base, 16)]
            r1 = rowslab[pl.ds(base + 16, 16)]
            idxbuf[pl.ds(0, 16)] = r0 + toff
            idxbuf[pl.ds(16, 16)] = r1 + toff
            pltpu.async_copy(hpp.at[idxbuf], gbuf, sem).wait()
            for j in range(K):
                sv = jnp.full((16,), wslab[base + j], _f32)
                for v in range(C // 16):
                    gbuf[j, pl.ds(v * 16, 16)] = gbuf[j, pl.ds(v * 16, 16)] * sv
            pltpu.sync_copy(gbuf, acc.at[colstage.at[cix]], add=True)
            return carry

        lax.fori_loop(0, NCH, chunk, 0)
        plsc.subcore_barrier()
        pltpu.sync_copy(acc.at[pl.ds(sid * RPT, RPT)],
                        out.at[pl.ds(toff + sid * RPT, RPT)])
        plsc.subcore_barrier()


BN_A = 1024


def _mm_body(x_ref, wc_ref, wg_ref, bc_ref, degp_ref, out_ref):
    xb = x_ref[...]
    wg_t = wg_ref[...].T
    m = [jnp.dot(wc_ref[k].T, wg_t, preferred_element_type=_f32)
         for k in range(3)]
    hb = jnp.dot(bc_ref[...], wg_t, preferred_element_type=_f32)
    pb = degp_ref[...]
    dis = lax.rsqrt(1.0 + pb[0, :, 0:1] + pb[1, :, 0:1])
    for t in range(T):
        acc = jnp.dot(xb[t], m[1], preferred_element_type=_f32) + hb
        if t > 0:
            acc = acc + jnp.dot(xb[t - 1], m[0], preferred_element_type=_f32)
        if t < T - 1:
            acc = acc + jnp.dot(xb[t + 1], m[2], preferred_element_type=_f32)
        out_ref[t] = acc * dis


_mm_call = pl.pallas_call(
    _mm_body,
    grid=(NP // BN_A,),
    in_specs=[
        pl.BlockSpec((T, BN_A, C), lambda i: (0, i, 0)),
        pl.BlockSpec((3, C, C), lambda i: (0, 0, 0)),
        pl.BlockSpec((C, C), lambda i: (0, 0)),
        pl.BlockSpec((1, C), lambda i: (0, 0)),
        pl.BlockSpec((2, BN_A, DL), lambda i: (0, i, 0)),
    ],
    out_specs=pl.BlockSpec((T, BN_A, C), lambda i: (0, i, 0)),
    out_shape=jax.ShapeDtypeStruct((T, NP, C), _f32),
)


def _bn_body(s_ref, h_ref, degp_ref, gamma_ref, beta_ref, out_ref):
    sb = s_ref[0]
    hb = h_ref[0]
    pb = degp_ref[...]
    dis = lax.rsqrt(1.0 + pb[0, :, 0:1] + pb[1, :, 0:1])
    o = dis * (sb + hb)
    mask = lax.broadcasted_iota(_i32, (NP, 1), 0) < N
    om = jnp.where(mask, o, 0.0)
    mu = jnp.sum(om, axis=0, keepdims=True) * (1.0 / N)
    d = jnp.where(mask, o - mu, 0.0)
    var = jnp.sum(d * d, axis=0, keepdims=True) * (1.0 / N)
    scale = gamma_ref[...] * lax.rsqrt(var + 1e-5)
    out_ref[0] = jnp.maximum((o - mu) * scale + beta_ref[...], 0.0)


_bn_call = pl.pallas_call(
    _bn_body,
    grid=(T,),
    in_specs=[
        pl.BlockSpec((1, NP, C), lambda t: (t, 0, 0)),
        pl.BlockSpec((1, NP, C), lambda t: (t, 0, 0)),
        pl.BlockSpec((2, NP, DL), lambda t: (0, 0, 0)),
        pl.BlockSpec((1, C), lambda t: (0, 0)),
        pl.BlockSpec((1, C), lambda t: (0, 0)),
    ],
    out_specs=pl.BlockSpec((1, NP, C), lambda t: (t, 0, 0)),
    out_shape=jax.ShapeDtypeStruct((T, NP, C), _f32),
)


BN_D = 1000


def _out_body(x_ref, w_ref, b_ref, out_ref):
    xb = x_ref[...]
    m = xb[0]
    for t in range(1, T):
        m = m + xb[t]
    m = m * (1.0 / T)
    out_ref[...] = jnp.dot(m, w_ref[...].T, preferred_element_type=_f32) \
        + b_ref[...]


_out_call = pl.pallas_call(
    _out_body,
    grid=(N // BN_D,),
    in_specs=[
        pl.BlockSpec((T, BN_D, C), lambda i: (0, i, 0)),
        pl.BlockSpec((C, C), lambda i: (0, 0)),
        pl.BlockSpec((1, C), lambda i: (0, 0)),
    ],
    out_specs=pl.BlockSpec((BN_D, C), lambda i: (i, 0)),
    out_shape=jax.ShapeDtypeStruct((N, C), _f32),
)


def kernel(x_seq, edge_index, edge_weight, l0_wc, l0_bc, l0_wg, l0_bg,
           l0_gamma, l0_beta, l1_wc, l1_bc, l1_wg, l1_bg, l1_gamma, l1_beta,
           out_w, out_b):
    row = edge_index[0]
    col = edge_index[1]
    w = edge_weight

    degp = _deg_kernel(col.reshape(NC * NS, DCH, DK),
                       w.reshape(NC * NS, DCH, DK))
    degp = degp.reshape(2, NP, DL)

    col3 = col.reshape(NS, NCH, K)
    x = jnp.zeros((T, NP, C), _f32).at[:, :N, :].set(x_seq)
    for (wc, bc, wg, gamma, beta) in (
            (l0_wc, l0_bc, l0_wg, l0_gamma, l0_beta),
            (l1_wc, l1_bc, l1_wg, l1_gamma, l1_beta)):
        wc_r = jnp.transpose(wc, (2, 0, 1))
        hpp = _mm_call(x, wc_r, wg, bc.reshape(1, C), degp)
        s = _spmm_kernel(hpp.reshape(T * NP, C), row, col3, w)
        x = _bn_call(s.reshape(T, NP, C), hpp, degp,
                     gamma.reshape(1, C), beta.reshape(1, C))
    return _out_call(x[:, :N, :], out_w, out_b.reshape(1, C))


# speedup vs baseline: 2.0671x; 2.0671x over previous
import functools

import jax
import jax.numpy as jnp
from jax import lax
from jax.experimental import pallas as pl
from jax.experimental.pallas import tpu as pltpu
from jax.experimental.pallas import tpu_sc as plsc

N = 10000
E = 320000
T = 8
C = 128
NP = 10240
NC = 2
NS = 16
RPT = NP // NS
TPS = T // NC

NB = 16
NSRC = 32
EPW = E // NSRC
RCAP = 10272
SCH = 1024
K = 32
DL = 16

_f32 = jnp.float32
_i32 = jnp.int32

_sc_mesh = plsc.VectorSubcoreMesh(
    core_axis_name="c", subcore_axis_name="s", num_cores=NC, num_subcores=NS)


def _extract(v0, v1, j):
    return v0[j] if j < 16 else v1[j - 16]


def _dyn_lane(v0, v1, j):
    acc = v0[0]
    for k in range(1, 16):
        acc = jnp.where(j == k, v0[k], acc)
    for k in range(16):
        acc = jnp.where(j == k + 16, v1[k], acc)
    return acc


@functools.partial(
    pl.kernel,
    out_type=jax.ShapeDtypeStruct((NC * NP, DL), _f32),
    mesh=_sc_mesh,
    scratch_types=[
        pltpu.VMEM((NSRC,), _i32),
        pltpu.VMEM((SCH,), _i32),
        pltpu.VMEM((SCH,), _f32),
        pltpu.VMEM((RPT, DL), _f32),
    ],
)
def _deg_kernel(bcl, bw, counts2, out, cntv, scl, sw, dacc):
    cid = lax.axis_index("c")
    b = lax.axis_index("s")
    pltpu.sync_copy(counts2.at[b], cntv)
    cv0 = cntv[pl.ds(0, 16)]
    cv1 = cntv[pl.ds(16, 16)]

    zv = jnp.zeros((DL,), _f32)

    def zr(i, carry):
        dacc[i, :] = zv
        return carry

    lax.fori_loop(0, RPT, zr, 0)

    cvsel = jnp.where(cid == 0, cv0, cv1)

    def sloop(sl, carry):
        s = cid * (NSRC // NC) + sl
        cnt = _dyn_lane(cvsel, cvsel, sl)
        roff = (b * NSRC + s) * RCAP
        nstage = (cnt + (SCH - 1)) // SCH

        def stage(si, carry1):
            off = roff + si * SCH
            pltpu.sync_copy(bcl.at[pl.ds(off, SCH)], scl)
            pltpu.sync_copy(bw.at[pl.ds(off, SCH)], sw)
            rem = jnp.minimum(cnt - si * SCH, SCH)
            nin = (rem + (K - 1)) // K

            def chunk(ci, carry2):
                base = ci * K
                c0 = scl[pl.ds(base, 16)]
                c1 = scl[pl.ds(base + 16, 16)]
                w0 = sw[pl.ds(base, 16)]
                w1 = sw[pl.ds(base + 16, 16)]
                for j in range(K):
                    cl = _extract(c0, c1, j)
                    wj = _extract(w0, w1, j)
                    dacc[cl, :] = dacc[cl, :] + jnp.full((DL,), wj, _f32)
                return carry2

            lax.fori_loop(0, nin, chunk, 0)
            return carry1

        lax.fori_loop(0, nstage, stage, 0)
        return carry

    lax.fori_loop(0, NSRC // NC, sloop, 0)

    pltpu.sync_copy(dacc, out.at[pl.ds(cid * NP + b * RPT, RPT)])


@functools.partial(
    pl.kernel,
    out_type=jax.ShapeDtypeStruct((T * NP, C), _f32),
    mesh=_sc_mesh,
    scratch_types=[
        pltpu.VMEM((NSRC,), _i32),
        pltpu.VMEM((SCH,), _i32),
        pltpu.VMEM((SCH,), _i32),
        pltpu.VMEM((SCH,), _f32),
        pltpu.VMEM((K,), _i32),
        pltpu.VMEM((K, C), _f32),
        pltpu.VMEM((RPT, C), _f32),
        pltpu.SemaphoreType.DMA,
    ],
)
def _spmm_kernel(hpp, brow, bcl, bw, counts2, out,
                 cntv, srow, scl, sw, idxbuf, gbuf, acc, sem):
    cid = lax.axis_index("c")
    b = lax.axis_index("s")
    pltpu.sync_copy(counts2.at[b], cntv)
    cv0 = cntv[pl.ds(0, 16)]
    cv1 = cntv[pl.ds(16, 16)]

    zv = jnp.zeros((16,), _f32)

    def tloop(tl, tcarry):
        t = cid * TPS + tl
        toff = t * NP

        def zr(i, carry):
            for v in range(C // 16):
                acc[i, pl.ds(v * 16, 16)] = zv
            return carry

        lax.fori_loop(0, RPT, zr, 0)

        def sloop(s, carry):
            cnt = _dyn_lane(cv0, cv1, s)
            roff = (b * NSRC + s) * RCAP
            nstage = (cnt + (SCH - 1)) // SCH

            def stage(si, carry1):
                off = roff + si * SCH
                pltpu.sync_copy(brow.at[pl.ds(off, SCH)], srow)
                pltpu.sync_copy(bcl.at[pl.ds(off, SCH)], scl)
                pltpu.sync_copy(bw.at[pl.ds(off, SCH)], sw)
                rem = jnp.minimum(cnt - si * SCH, SCH)
                nin = (rem + (K - 1)) // K

                def chunk(ci, carry2):
                    base = ci * K
                    idxbuf[pl.ds(0, 16)] = srow[pl.ds(base, 16)] + toff
                    idxbuf[pl.ds(16, 16)] = srow[pl.ds(base + 16, 16)] + toff
                    pltpu.async_copy(hpp.at[idxbuf], gbuf, sem).wait()
                    c0 = scl[pl.ds(base, 16)]
                    c1 = scl[pl.ds(base + 16, 16)]
                    w0 = sw[pl.ds(base, 16)]
                    w1 = sw[pl.ds(base + 16, 16)]
                    for j in range(K):
                        cl = _extract(c0, c1, j)
                        sv = jnp.full((16,), _extract(w0, w1, j), _f32)
                        for v in range(C // 16):
                            acc[cl, pl.ds(v * 16, 16)] = (
                                acc[cl, pl.ds(v * 16, 16)]
                                + gbuf[j, pl.ds(v * 16, 16)] * sv)
                    return carry2

                lax.fori_loop(0, nin, chunk, 0)
                return carry1

            lax.fori_loop(0, nstage, stage, 0)
            return carry

        lax.fori_loop(0, NSRC, sloop, 0)
        pltpu.sync_copy(acc, out.at[pl.ds(toff + b * RPT, RPT)])
        return tcarry

    lax.fori_loop(0, TPS, tloop, 0)


BN_A = 1024


def _mm_body(x_ref, wc_ref, wg_ref, bc_ref, degp_ref, out_ref):
    xb = x_ref[...]
    wg_t = wg_ref[...].T
    m = [jnp.dot(wc_ref[k].T, wg_t, preferred_element_type=_f32)
         for k in range(3)]
    hb = jnp.dot(bc_ref[...], wg_t, preferred_element_type=_f32)
    pb = degp_ref[...]
    dis = lax.rsqrt(1.0 + pb[0, :, 0:1] + pb[1, :, 0:1])
    for t in range(T):
        acc = jnp.dot(xb[t], m[1], preferred_element_type=_f32) + hb
        if t > 0:
            acc = acc + jnp.dot(xb[t - 1], m[0], preferred_element_type=_f32)
        if t < T - 1:
            acc = acc + jnp.dot(xb[t + 1], m[2], preferred_element_type=_f32)
        out_ref[t] = acc * dis


_mm_call = pl.pallas_call(
    _mm_body,
    grid=(NP // BN_A,),
    in_specs=[
        pl.BlockSpec((T, BN_A, C), lambda i: (0, i, 0)),
        pl.BlockSpec((3, C, C), lambda i: (0, 0, 0)),
        pl.BlockSpec((C, C), lambda i: (0, 0)),
        pl.BlockSpec((1, C), lambda i: (0, 0)),
        pl.BlockSpec((2, BN_A, DL), lambda i: (0, i, 0)),
    ],
    out_specs=pl.BlockSpec((T, BN_A, C), lambda i: (0, i, 0)),
    out_shape=jax.ShapeDtypeStruct((T, NP, C), _f32),
)


def _bn_body(s_ref, h_ref, degp_ref, gamma_ref, beta_ref, out_ref):
    sb = s_ref[0]
    hb = h_ref[0]
    pb = degp_ref[...]
    dis = lax.rsqrt(1.0 + pb[0, :, 0:1] + pb[1, :, 0:1])
    o = dis * (sb + hb)
    mask = lax.broadcasted_iota(_i32, (NP, 1), 0) < N
    om = jnp.where(mask, o, 0.0)
    mu = jnp.sum(om, axis=0, keepdims=True) * (1.0 / N)
    d = jnp.where(mask, o - mu, 0.0)
    var = jnp.sum(d * d, axis=0, keepdims=True) * (1.0 / N)
    scale = gamma_ref[...] * lax.rsqrt(var + 1e-5)
    out_ref[0] = jnp.maximum((o - mu) * scale + beta_ref[...], 0.0)


_bn_call = pl.pallas_call(
    _bn_body,
    grid=(T,),
    in_specs=[
        pl.BlockSpec((1, NP, C), lambda t: (t, 0, 0)),
        pl.BlockSpec((1, NP, C), lambda t: (t, 0, 0)),
        pl.BlockSpec((2, NP, DL), lambda t: (0, 0, 0)),
        pl.BlockSpec((1, C), lambda t: (0, 0)),
        pl.BlockSpec((1, C), lambda t: (0, 0)),
    ],
    out_specs=pl.BlockSpec((1, NP, C), lambda t: (t, 0, 0)),
    out_shape=jax.ShapeDtypeStruct((T, NP, C), _f32),
)


BN_D = 1000


def _out_body(x_ref, w_ref, b_ref, out_ref):
    xb = x_ref[...]
    m = xb[0]
    for t in range(1, T):
        m = m + xb[t]
    m = m * (1.0 / T)
    out_ref[...] = jnp.dot(m, w_ref[...].T, preferred_element_type=_f32) \
        + b_ref[...]


_out_call = pl.pallas_call(
    _out_body,
    grid=(N // BN_D,),
    in_specs=[
        pl.BlockSpec((T, BN_D, C), lambda i: (0, i, 0)),
        pl.BlockSpec((C, C), lambda i: (0, 0)),
        pl.BlockSpec((1, C), lambda i: (0, 0)),
    ],
    out_specs=pl.BlockSpec((BN_D, C), lambda i: (i, 0)),
    out_shape=jax.ShapeDtypeStruct((N, C), _f32),
)


def _bucketize(row, col, w):
    bkt = col // RPT
    key = bkt * NSRC + (jnp.arange(E, dtype=_i32) // EPW)
    order = jnp.argsort(key)
    keys = key[order]
    counts = jnp.zeros((NB * NSRC,), _i32).at[keys].add(1)
    starts = jnp.concatenate(
        [jnp.zeros((1,), _i32), jnp.cumsum(counts)[:-1].astype(_i32)])
    rank = jnp.arange(E, dtype=_i32) - starts[keys]
    dest = keys * RCAP + rank
    brow = jnp.zeros((NB * NSRC * RCAP,), _i32).at[dest].set(row[order])
    bcl = jnp.zeros((NB * NSRC * RCAP,), _i32).at[dest].set(
        col[order] - (keys // NSRC) * RPT)
    bw = jnp.zeros((NB * NSRC * RCAP,), _f32).at[dest].set(w[order])
    return brow, bcl, bw, counts.reshape(NB, NSRC)


def kernel(x_seq, edge_index, edge_weight, l0_wc, l0_bc, l0_wg, l0_bg,
           l0_gamma, l0_beta, l1_wc, l1_bc, l1_wg, l1_bg, l1_gamma, l1_beta,
           out_w, out_b):
    row = edge_index[0]
    col = edge_index[1]
    w = edge_weight

    brow, bcl, bw, counts2 = _bucketize(row, col, w)
    degp = _deg_kernel(bcl, bw, counts2).reshape(2, NP, DL)

    x = jnp.zeros((T, NP, C), _f32).at[:, :N, :].set(x_seq)
    for (wc, bc, wg, gamma, beta) in (
            (l0_wc, l0_bc, l0_wg, l0_gamma, l0_beta),
            (l1_wc, l1_bc, l1_wg, l1_gamma, l1_beta)):
        wc_r = jnp.transpose(wc, (2, 0, 1))
        hpp = _mm_call(x, wc_r, wg, bc.reshape(1, C), degp)
        s = _spmm_kernel(hpp.reshape(T * NP, C), brow, bcl, bw, counts2)
        x = _bn_call(s.reshape(T, NP, C), hpp, degp,
                     gamma.reshape(1, C), beta.reshape(1, C))
    return _out_call(x[:, :N, :], out_w, out_b.reshape(1, C))
